# baseline (device time: 40083 ns/iter reference)
import jax
import jax.numpy as jnp
from jax import lax
from jax.experimental import pallas as pl
from jax.experimental.pallas import tpu as pltpu

N_LAYERS = 3
N_CHUNKS = 8


def kernel(x, Win0, Wout0, Win1, Wout1, Win2, Wout2):
    b, d_y = x.shape
    _, h_x = Win0.shape
    bc = b // N_CHUNKS
    n_slots = N_LAYERS * N_CHUNKS

    def body(x_ref, win0_ref, wout0_ref, win1_ref, wout1_ref, win2_ref,
             wout2_ref, out_ref,
             h_send, h_recv, o_send, o_recv, ph_f32, po_f32,
             h_send_sems, h_recv_sems, o_send_sems, o_recv_sems):
        my_x = lax.axis_index("x")
        my_y = lax.axis_index("y")
        y_peer = (my_x, 1 - my_y)
        x_peer = (1 - my_x, my_y)

        barrier_sem = pltpu.get_barrier_semaphore()
        for nbr in (y_peer, x_peer):
            pl.semaphore_signal(barrier_sem, inc=1, device_id=nbr,
                                device_id_type=pl.DeviceIdType.MESH)
        pl.semaphore_wait(barrier_sem, 2)

        wins = (win0_ref, win1_ref, win2_ref)
        wouts = (wout0_ref, wout1_ref, wout2_ref)

        def o_rdma(s):
            return pltpu.make_async_remote_copy(
                src_ref=o_send.at[s], dst_ref=o_recv.at[s],
                send_sem=o_send_sems.at[s], recv_sem=o_recv_sems.at[s],
                device_id=x_peer, device_id_type=pl.DeviceIdType.MESH,
            )

        def h_rdma(s):
            return pltpu.make_async_remote_copy(
                src_ref=h_send.at[s], dst_ref=h_recv.at[s],
                send_sem=h_send_sems.at[s], recv_sem=h_recv_sems.at[s],
                device_id=y_peer, device_id_type=pl.DeviceIdType.MESH,
            )

        def step_a(k, c):
            s = k * N_CHUNKS + c
            if k == 0:
                x_c = x_ref[pl.ds(c * bc, bc), :]
            else:
                sp = (k - 1) * N_CHUNKS + c
                o_rdma(sp).wait()
                x_c = po_f32[c, :, :] + o_recv[sp, :, :].astype(jnp.float32)
            ph = jnp.dot(
                x_c, wins[k][:, :], preferred_element_type=jnp.float32)
            ph_f32[c, :, :] = ph
            h_send[s, :, :] = ph.astype(jnp.bfloat16)
            h_rdma(s).start()

        def step_b(k, c):
            s = k * N_CHUNKS + c
            h_rdma(s).wait()
            h = jnp.maximum(
                ph_f32[c, :, :] + h_recv[s, :, :].astype(jnp.float32), 0.0)
            po = jnp.dot(
                h, wouts[k][:, :], preferred_element_type=jnp.float32)
            po_f32[c, :, :] = po
            o_send[s, :, :] = po.astype(jnp.bfloat16)
            o_rdma(s).start()

        for c in range(N_CHUNKS):
            step_a(0, c)
        for k in range(N_LAYERS):
            for c in range(N_CHUNKS):
                step_b(k, c)
                if k + 1 < N_LAYERS and c >= 1:
                    step_a(k + 1, c - 1)
            if k + 1 < N_LAYERS:
                step_a(k + 1, N_CHUNKS - 1)

        for c in range(N_CHUNKS):
            s = (N_LAYERS - 1) * N_CHUNKS + c
            o_rdma(s).wait()
            out_ref[pl.ds(c * bc, bc), :] = (
                po_f32[c, :, :] + o_recv[s, :, :].astype(jnp.float32))

    return pl.pallas_call(
        body,
        out_shape=jax.ShapeDtypeStruct((b, d_y), jnp.float32),
        in_specs=[pl.BlockSpec(memory_space=pltpu.VMEM)] * 7,
        out_specs=pl.BlockSpec(memory_space=pltpu.VMEM),
        scratch_shapes=[
            pltpu.VMEM((n_slots, bc, h_x), jnp.bfloat16),
            pltpu.VMEM((n_slots, bc, h_x), jnp.bfloat16),
            pltpu.VMEM((n_slots, bc, d_y), jnp.bfloat16),
            pltpu.VMEM((n_slots, bc, d_y), jnp.bfloat16),
            pltpu.VMEM((N_CHUNKS, bc, h_x), jnp.float32),
            pltpu.VMEM((N_CHUNKS, bc, d_y), jnp.float32),
            pltpu.SemaphoreType.DMA((n_slots,)),
            pltpu.SemaphoreType.DMA((n_slots,)),
            pltpu.SemaphoreType.DMA((n_slots,)),
            pltpu.SemaphoreType.DMA((n_slots,)),
        ],
        compiler_params=pltpu.CompilerParams(collective_id=0),
    )(x, Win0, Wout0, Win1, Wout1, Win2, Wout2)


# device time: 38626 ns/iter; 1.0377x vs baseline; 1.0377x over previous
import jax
import jax.numpy as jnp
from jax import lax
from jax.experimental import pallas as pl
from jax.experimental.pallas import tpu as pltpu

N_LAYERS = 3
N_CHUNKS = 2


def kernel(x, Win0, Wout0, Win1, Wout1, Win2, Wout2):
    b, d_y = x.shape
    _, h_x = Win0.shape
    bc = b // N_CHUNKS
    n_slots = N_LAYERS * N_CHUNKS

    def body(x_ref, win0_ref, wout0_ref, win1_ref, wout1_ref, win2_ref,
             wout2_ref, out_ref,
             h_send, h_recv, o_send, o_recv, ph_f32, po_f32,
             h_send_sems, h_recv_sems, o_send_sems, o_recv_sems):
        my_x = lax.axis_index("x")
        my_y = lax.axis_index("y")
        y_peer = (my_x, 1 - my_y)
        x_peer = (1 - my_x, my_y)

        barrier_sem = pltpu.get_barrier_semaphore()
        for nbr in (y_peer, x_peer):
            pl.semaphore_signal(barrier_sem, inc=1, device_id=nbr,
                                device_id_type=pl.DeviceIdType.MESH)
        pl.semaphore_wait(barrier_sem, 2)

        wins = (win0_ref, win1_ref, win2_ref)
        wouts = (wout0_ref, wout1_ref, wout2_ref)

        def o_rdma(s):
            return pltpu.make_async_remote_copy(
                src_ref=o_send.at[s], dst_ref=o_recv.at[s],
                send_sem=o_send_sems.at[s], recv_sem=o_recv_sems.at[s],
                device_id=x_peer, device_id_type=pl.DeviceIdType.MESH,
            )

        def h_rdma(s):
            return pltpu.make_async_remote_copy(
                src_ref=h_send.at[s], dst_ref=h_recv.at[s],
                send_sem=h_send_sems.at[s], recv_sem=h_recv_sems.at[s],
                device_id=y_peer, device_id_type=pl.DeviceIdType.MESH,
            )

        def step_a(k, c):
            s = k * N_CHUNKS + c
            if k == 0:
                x_c = x_ref[pl.ds(c * bc, bc), :]
            else:
                sp = (k - 1) * N_CHUNKS + c
                o_rdma(sp).wait()
                x_c = po_f32[c, :, :] + o_recv[sp, :, :].astype(jnp.float32)
            ph = jnp.dot(
                x_c, wins[k][:, :], preferred_element_type=jnp.float32)
            ph_f32[c, :, :] = ph
            h_send[s, :, :] = ph.astype(jnp.bfloat16)
            h_rdma(s).start()

        def step_b(k, c):
            s = k * N_CHUNKS + c
            h_rdma(s).wait()
            h = jnp.maximum(
                ph_f32[c, :, :] + h_recv[s, :, :].astype(jnp.float32), 0.0)
            po = jnp.dot(
                h, wouts[k][:, :], preferred_element_type=jnp.float32)
            po_f32[c, :, :] = po
            o_send[s, :, :] = po.astype(jnp.bfloat16)
            o_rdma(s).start()

        for c in range(N_CHUNKS):
            step_a(0, c)
        for k in range(N_LAYERS):
            for c in range(N_CHUNKS):
                step_b(k, c)
                if k + 1 < N_LAYERS and c >= 1:
                    step_a(k + 1, c - 1)
            if k + 1 < N_LAYERS:
                step_a(k + 1, N_CHUNKS - 1)

        for c in range(N_CHUNKS):
            s = (N_LAYERS - 1) * N_CHUNKS + c
            o_rdma(s).wait()
            out_ref[pl.ds(c * bc, bc), :] = (
                po_f32[c, :, :] + o_recv[s, :, :].astype(jnp.float32))

    return pl.pallas_call(
        body,
        out_shape=jax.ShapeDtypeStruct((b, d_y), jnp.float32),
        in_specs=[pl.BlockSpec(memory_space=pltpu.VMEM)] * 7,
        out_specs=pl.BlockSpec(memory_space=pltpu.VMEM),
        scratch_shapes=[
            pltpu.VMEM((n_slots, bc, h_x), jnp.bfloat16),
            pltpu.VMEM((n_slots, bc, h_x), jnp.bfloat16),
            pltpu.VMEM((n_slots, bc, d_y), jnp.bfloat16),
            pltpu.VMEM((n_slots, bc, d_y), jnp.bfloat16),
            pltpu.VMEM((N_CHUNKS, bc, h_x), jnp.float32),
            pltpu.VMEM((N_CHUNKS, bc, d_y), jnp.float32),
            pltpu.SemaphoreType.DMA((n_slots,)),
            pltpu.SemaphoreType.DMA((n_slots,)),
            pltpu.SemaphoreType.DMA((n_slots,)),
            pltpu.SemaphoreType.DMA((n_slots,)),
        ],
        compiler_params=pltpu.CompilerParams(collective_id=0),
    )(x, Win0, Wout0, Win1, Wout1, Win2, Wout2)


# device time: 35236 ns/iter; 1.1376x vs baseline; 1.0962x over previous
import jax
import jax.numpy as jnp
from jax import lax
from jax.experimental import pallas as pl
from jax.experimental.pallas import tpu as pltpu

N_LAYERS = 3
N_CHUNKS = 4


def kernel(x, Win0, Wout0, Win1, Wout1, Win2, Wout2):
    b, d_y = x.shape
    _, h_x = Win0.shape
    bc = b // N_CHUNKS
    n_slots = N_LAYERS * N_CHUNKS

    def body(x_ref, win0_ref, wout0_ref, win1_ref, wout1_ref, win2_ref,
             wout2_ref, out_ref,
             h_send, h_recv, o_send, o_recv, ph_f32, po_f32,
             x_v, win_v, wout_v, load_sems,
             h_send_sems, h_recv_sems, o_send_sems, o_recv_sems):
        my_x = lax.axis_index("x")
        my_y = lax.axis_index("y")
        y_peer = (my_x, 1 - my_y)
        x_peer = (1 - my_x, my_y)

        win_hbm = (win0_ref, win1_ref, win2_ref)
        wout_hbm = (wout0_ref, wout1_ref, wout2_ref)
        x_copy = pltpu.make_async_copy(x_ref, x_v, load_sems.at[0])
        x_copy.start()
        win_copies = []
        wout_copies = []
        for k in range(N_LAYERS):
            wc = pltpu.make_async_copy(win_hbm[k], win_v.at[k],
                                       load_sems.at[1 + k])
            wc.start()
            win_copies.append(wc)
            oc = pltpu.make_async_copy(wout_hbm[k], wout_v.at[k],
                                       load_sems.at[1 + N_LAYERS + k])
            oc.start()
            wout_copies.append(oc)

        barrier_sem = pltpu.get_barrier_semaphore()
        for nbr in (y_peer, x_peer):
            pl.semaphore_signal(barrier_sem, inc=1, device_id=nbr,
                                device_id_type=pl.DeviceIdType.MESH)
        pl.semaphore_wait(barrier_sem, 2)

        def o_rdma(s):
            return pltpu.make_async_remote_copy(
                src_ref=o_send.at[s], dst_ref=o_recv.at[s],
                send_sem=o_send_sems.at[s], recv_sem=o_recv_sems.at[s],
                device_id=x_peer, device_id_type=pl.DeviceIdType.MESH,
            )

        def h_rdma(s):
            return pltpu.make_async_remote_copy(
                src_ref=h_send.at[s], dst_ref=h_recv.at[s],
                send_sem=h_send_sems.at[s], recv_sem=h_recv_sems.at[s],
                device_id=y_peer, device_id_type=pl.DeviceIdType.MESH,
            )

        def step_a(k, c):
            s = k * N_CHUNKS + c
            if c == 0:
                win_copies[k].wait()
                if k == 0:
                    x_copy.wait()
            if k == 0:
                x_c = x_v[pl.ds(c * bc, bc), :]
            else:
                sp = (k - 1) * N_CHUNKS + c
                o_rdma(sp).wait()
                x_c = po_f32[c, :, :] + o_recv[sp, :, :].astype(jnp.float32)
            ph = jnp.dot(
                x_c, win_v[k, :, :], preferred_element_type=jnp.float32)
            ph_f32[c, :, :] = ph
            h_send[s, :, :] = ph.astype(jnp.bfloat16)
            h_rdma(s).start()

        def step_b(k, c):
            s = k * N_CHUNKS + c
            if c == 0:
                wout_copies[k].wait()
            h_rdma(s).wait()
            h = jnp.maximum(
                ph_f32[c, :, :] + h_recv[s, :, :].astype(jnp.float32), 0.0)
            po = jnp.dot(
                h, wout_v[k, :, :], preferred_element_type=jnp.float32)
            po_f32[c, :, :] = po
            o_send[s, :, :] = po.astype(jnp.bfloat16)
            o_rdma(s).start()

        for c in range(N_CHUNKS):
            step_a(0, c)
        for k in range(N_LAYERS):
            for c in range(N_CHUNKS):
                step_b(k, c)
                if k + 1 < N_LAYERS and c >= 1:
                    step_a(k + 1, c - 1)
            if k + 1 < N_LAYERS:
                step_a(k + 1, N_CHUNKS - 1)

        for c in range(N_CHUNKS):
            s = (N_LAYERS - 1) * N_CHUNKS + c
            o_rdma(s).wait()
            out_ref[pl.ds(c * bc, bc), :] = (
                po_f32[c, :, :] + o_recv[s, :, :].astype(jnp.float32))

    return pl.pallas_call(
        body,
        out_shape=jax.ShapeDtypeStruct((b, d_y), jnp.float32),
        in_specs=[pl.BlockSpec(memory_space=pl.ANY)] * 7,
        out_specs=pl.BlockSpec(memory_space=pltpu.VMEM),
        scratch_shapes=[
            pltpu.VMEM((n_slots, bc, h_x), jnp.bfloat16),
            pltpu.VMEM((n_slots, bc, h_x), jnp.bfloat16),
            pltpu.VMEM((n_slots, bc, d_y), jnp.bfloat16),
            pltpu.VMEM((n_slots, bc, d_y), jnp.bfloat16),
            pltpu.VMEM((N_CHUNKS, bc, h_x), jnp.float32),
            pltpu.VMEM((N_CHUNKS, bc, d_y), jnp.float32),
            pltpu.VMEM((b, d_y), jnp.float32),
            pltpu.VMEM((N_LAYERS, d_y, h_x), jnp.float32),
            pltpu.VMEM((N_LAYERS, h_x, d_y), jnp.float32),
            pltpu.SemaphoreType.DMA((1 + 2 * N_LAYERS,)),
            pltpu.SemaphoreType.DMA((n_slots,)),
            pltpu.SemaphoreType.DMA((n_slots,)),
            pltpu.SemaphoreType.DMA((n_slots,)),
            pltpu.SemaphoreType.DMA((n_slots,)),
        ],
        compiler_params=pltpu.CompilerParams(collective_id=0),
    )(x, Win0, Wout0, Win1, Wout1, Win2, Wout2)


# device time: 34846 ns/iter; 1.1503x vs baseline; 1.0112x over previous
import jax
import jax.numpy as jnp
from jax import lax
from jax.experimental import pallas as pl
from jax.experimental.pallas import tpu as pltpu

N_LAYERS = 3
N_CHUNKS = 4


def kernel(x, Win0, Wout0, Win1, Wout1, Win2, Wout2):
    b, d_y = x.shape
    _, h_x = Win0.shape
    bc = b // N_CHUNKS
    n_slots = N_LAYERS * N_CHUNKS

    def body(x_ref, win0_ref, wout0_ref, win1_ref, wout1_ref, win2_ref,
             wout2_ref, out_ref,
             h_send, h_recv, o_send, o_recv, ph_f32, po_f32,
             h_send_sems, h_recv_sems, o_send_sems, o_recv_sems):
        my_x = lax.axis_index("x")
        my_y = lax.axis_index("y")
        y_peer = (my_x, 1 - my_y)
        x_peer = (1 - my_x, my_y)

        wins = (win0_ref, win1_ref, win2_ref)
        wouts = (wout0_ref, wout1_ref, wout2_ref)

        barrier_sem = pltpu.get_barrier_semaphore()
        for nbr in (y_peer, x_peer):
            pl.semaphore_signal(barrier_sem, inc=1, device_id=nbr,
                                device_id_type=pl.DeviceIdType.MESH)
        pl.semaphore_wait(barrier_sem, 2)

        def o_rdma(s):
            return pltpu.make_async_remote_copy(
                src_ref=o_send.at[s], dst_ref=o_recv.at[s],
                send_sem=o_send_sems.at[s], recv_sem=o_recv_sems.at[s],
                device_id=x_peer, device_id_type=pl.DeviceIdType.MESH,
            )

        def h_rdma(s):
            return pltpu.make_async_remote_copy(
                src_ref=h_send.at[s], dst_ref=h_recv.at[s],
                send_sem=h_send_sems.at[s], recv_sem=h_recv_sems.at[s],
                device_id=y_peer, device_id_type=pl.DeviceIdType.MESH,
            )

        def step_a(k, c):
            s = k * N_CHUNKS + c
            if k == 0:
                x_c = x_ref[pl.ds(c * bc, bc), :]
            else:
                sp = (k - 1) * N_CHUNKS + c
                o_rdma(sp).wait()
                x_c = po_f32[c, :, :] + o_recv[sp, :, :].astype(jnp.float32)
            ph = jnp.dot(
                x_c, wins[k][:, :], preferred_element_type=jnp.float32)
            ph_f32[c, :, :] = ph
            h_send[s, :, :] = ph.astype(jnp.bfloat16)
            h_rdma(s).start()

        def step_b(k, c):
            s = k * N_CHUNKS + c
            h_rdma(s).wait()
            h = jnp.maximum(
                ph_f32[c, :, :] + h_recv[s, :, :].astype(jnp.float32), 0.0)
            po = jnp.dot(
                h, wouts[k][:, :], preferred_element_type=jnp.float32)
            po_f32[c, :, :] = po
            o_send[s, :, :] = po.astype(jnp.bfloat16)
            o_rdma(s).start()

        for c in range(N_CHUNKS):
            step_a(0, c)
        for k in range(N_LAYERS):
            for c in range(N_CHUNKS):
                step_b(k, c)
                if k + 1 < N_LAYERS and c >= 1:
                    step_a(k + 1, c - 1)
            if k + 1 < N_LAYERS:
                step_a(k + 1, N_CHUNKS - 1)

        for c in range(N_CHUNKS):
            s = (N_LAYERS - 1) * N_CHUNKS + c
            o_rdma(s).wait()
            out_ref[pl.ds(c * bc, bc), :] = (
                po_f32[c, :, :] + o_recv[s, :, :].astype(jnp.float32))

    return pl.pallas_call(
        body,
        out_shape=jax.ShapeDtypeStruct((b, d_y), jnp.float32),
        in_specs=[pl.BlockSpec(memory_space=pltpu.VMEM)] * 7,
        out_specs=pl.BlockSpec(memory_space=pltpu.VMEM),
        scratch_shapes=[
            pltpu.VMEM((n_slots, bc, h_x), jnp.bfloat16),
            pltpu.VMEM((n_slots, bc, h_x), jnp.bfloat16),
            pltpu.VMEM((n_slots, bc, d_y), jnp.bfloat16),
            pltpu.VMEM((n_slots, bc, d_y), jnp.bfloat16),
            pltpu.VMEM((N_CHUNKS, bc, h_x), jnp.float32),
            pltpu.VMEM((N_CHUNKS, bc, d_y), jnp.float32),
            pltpu.SemaphoreType.DMA((n_slots,)),
            pltpu.SemaphoreType.DMA((n_slots,)),
            pltpu.SemaphoreType.DMA((n_slots,)),
            pltpu.SemaphoreType.DMA((n_slots,)),
        ],
        compiler_params=pltpu.CompilerParams(collective_id=0),
    )(x, Win0, Wout0, Win1, Wout1, Win2, Wout2)
